# baseline (device time: 15216 ns/iter reference)
import jax
import jax.numpy as jnp
from jax import lax
from jax.experimental import pallas as pl
from jax.experimental.pallas import tpu as pltpu

N_DEV = 4


def kernel(t):
    m, n = t.shape

    def body(x_ref, out_ref, stage_ref, comm_ref, res_ref,
             send_sems, recv_sems, out_copy_sem):
        my = lax.axis_index("i")


        stage_ref[:, :] = x_ref[:, :].astype(jnp.bfloat16)

        rdmas = {}
        for k in (2, 1, 3):
            rdma = pltpu.make_async_remote_copy(
                src_ref=stage_ref,
                dst_ref=comm_ref.at[k - 1],
                send_sem=send_sems.at[k - 1],
                recv_sem=recv_sems.at[k - 1],
                device_id=((my + k) % N_DEV,),
                device_id_type=pl.DeviceIdType.MESH,
            )
            rdma.start()
            rdmas[k] = rdma

        s = x_ref[:, :]
        for k in (1, 3, 2):
            rdmas[k].wait_recv()
            s = s + comm_ref[k - 1, :, :].astype(jnp.float32)
        r = jnp.maximum(s, 0.0)
        res_ref[:, :] = (jnp.tanh(s) * s * s + r * r * r).astype(jnp.bfloat16)

        out_copy = pltpu.make_async_copy(res_ref, out_ref, out_copy_sem)
        out_copy.start()
        out_copy.wait()

        for k in (1, 2, 3):
            rdmas[k].wait_send()

    return pl.pallas_call(
        body,
        out_shape=jax.ShapeDtypeStruct((m, n), jnp.bfloat16),
        in_specs=[pl.BlockSpec(memory_space=pltpu.VMEM)],
        out_specs=pl.BlockSpec(memory_space=pl.ANY),
        scratch_shapes=[
            pltpu.VMEM((m, n), jnp.bfloat16),
            pltpu.VMEM((N_DEV - 1, m, n), jnp.bfloat16),
            pltpu.VMEM((m, n), jnp.bfloat16),
            pltpu.SemaphoreType.DMA((N_DEV - 1,)),
            pltpu.SemaphoreType.DMA((N_DEV - 1,)),
            pltpu.SemaphoreType.DMA(()),
        ],
    )(t)


# device time: 12474 ns/iter; 1.2198x vs baseline; 1.2198x over previous
import jax
import jax.numpy as jnp
from jax import lax
from jax.experimental import pallas as pl
from jax.experimental.pallas import tpu as pltpu

N_DEV = 4


def kernel(t):
    m, n = t.shape

    def body(x_ref, out_ref, stage_ref, comm_ref, send_sems, recv_sems):
        my = lax.axis_index("i")

        barrier_sem = pltpu.get_barrier_semaphore()
        for k in range(1, N_DEV):
            pl.semaphore_signal(
                barrier_sem, inc=1,
                device_id=((my + k) % N_DEV,),
                device_id_type=pl.DeviceIdType.MESH,
            )

        stage_ref[:, :] = x_ref[:, :].astype(jnp.bfloat16)

        pl.semaphore_wait(barrier_sem, N_DEV - 1)

        rdmas = {}
        for k in (1, 3, 2):
            rdma = pltpu.make_async_remote_copy(
                src_ref=stage_ref,
                dst_ref=comm_ref.at[k - 1],
                send_sem=send_sems.at[k - 1],
                recv_sem=recv_sems.at[k - 1],
                device_id=((my + k) % N_DEV,),
                device_id_type=pl.DeviceIdType.MESH,
            )
            rdma.start()
            rdmas[k] = rdma

        s = x_ref[:, :]
        for k in (1, 3, 2):
            rdmas[k].wait_recv()
            s = s + comm_ref[k - 1, :, :].astype(jnp.float32)
        r = jnp.maximum(s, 0.0)
        out_ref[:, :] = (jnp.tanh(s) * s * s + r * r * r).astype(jnp.bfloat16)

        for k in (1, 2, 3):
            rdmas[k].wait_send()

    return pl.pallas_call(
        body,
        out_shape=jax.ShapeDtypeStruct((m, n), jnp.bfloat16),
        in_specs=[pl.BlockSpec(memory_space=pltpu.VMEM)],
        out_specs=pl.BlockSpec(memory_space=pltpu.VMEM),
        scratch_shapes=[
            pltpu.VMEM((m, n), jnp.bfloat16),
            pltpu.VMEM((N_DEV - 1, m, n), jnp.bfloat16),
            pltpu.SemaphoreType.DMA((N_DEV - 1,)),
            pltpu.SemaphoreType.DMA((N_DEV - 1,)),
        ],
        compiler_params=pltpu.CompilerParams(collective_id=0),
    )(t)
